# baseline (speedup 1.0000x reference)
import functools

import jax
import jax.numpy as jnp
from jax.experimental import pallas as pl

N = 10000
E = 320000
H = 128
NRBF = 50
NBLK = 3
CUTOFF = 5.0

EB = 5000


_LOG2 = 0.6931471805599453


def _softplus(x):
    return jnp.maximum(x, 0.0) + jnp.log(1.0 + jnp.exp(-jnp.abs(x))) - _LOG2


def _edge_msg_body(d2_ref, hg_ref, f1_wt_ref, f1_b_ref, f2_wt_ref, f2_b_ref,
                   out_ref):
    d2 = d2_ref[...]
    d = jnp.sqrt(d2 + 1e-12)
    delta = CUTOFF / (NRBF - 1)
    offs = jax.lax.broadcasted_iota(jnp.int32, (EB, NRBF), 1).astype(
        jnp.float32) * delta
    coeff = -0.5 / (delta * delta)
    rbf = jnp.exp(coeff * (d - offs) ** 2)
    w = _softplus(jnp.dot(rbf, f1_wt_ref[...],
                          preferred_element_type=jnp.float32) + f1_b_ref[...])
    w = _softplus(jnp.dot(w, f2_wt_ref[...],
                          preferred_element_type=jnp.float32) + f2_b_ref[...])
    c = 0.5 * (jnp.cos(d * (jnp.pi / CUTOFF)) + 1.0)
    c = jnp.where(d < CUTOFF, c, 0.0)
    out_ref[...] = hg_ref[...] * w * c


@jax.jit
def _edge_msg(d2, hg, f1_wt, f1_b, f2_wt, f2_b):
    grid = (E // EB,)
    return pl.pallas_call(
        _edge_msg_body,
        grid=grid,
        in_specs=[
            pl.BlockSpec((EB, 1), lambda i: (i, 0)),
            pl.BlockSpec((EB, H), lambda i: (i, 0)),
            pl.BlockSpec((NRBF, H), lambda i: (0, 0)),
            pl.BlockSpec((1, H), lambda i: (0, 0)),
            pl.BlockSpec((H, H), lambda i: (0, 0)),
            pl.BlockSpec((1, H), lambda i: (0, 0)),
        ],
        out_specs=pl.BlockSpec((EB, H), lambda i: (i, 0)),
        out_shape=jax.ShapeDtypeStruct((E, H), jnp.float32),
    )(d2, hg, f1_wt, f1_b, f2_wt, f2_b)


def _node_lin_body(x_ref, wt_ref, out_ref):
    out_ref[...] = jnp.dot(x_ref[...], wt_ref[...],
                           preferred_element_type=jnp.float32)


@jax.jit
def _node_lin(x, wt):
    return pl.pallas_call(
        _node_lin_body,
        out_shape=jax.ShapeDtypeStruct((N, H), jnp.float32),
    )(x, wt)


def _node_update_body(x_ref, agg_ref, lin2_wt_ref, lin2_b_ref, lin_wt_ref,
                      lin_b_ref, out_ref):
    h = jnp.dot(agg_ref[...], lin2_wt_ref[...],
                preferred_element_type=jnp.float32) + lin2_b_ref[...]
    h = jnp.tanh(h)
    h = jnp.dot(h, lin_wt_ref[...],
                preferred_element_type=jnp.float32) + lin_b_ref[...]
    out_ref[...] = x_ref[...] + h


@jax.jit
def _node_update(x, agg, lin2_wt, lin2_b, lin_wt, lin_b):
    return pl.pallas_call(
        _node_update_body,
        out_shape=jax.ShapeDtypeStruct((N, H), jnp.float32),
    )(x, agg, lin2_wt, lin2_b, lin_wt, lin_b)


def kernel(z, pos, edge_index, emb, lin1_W, f1_W, f1_b, f2_W, f2_b, lin2_W,
           lin2_b, lin_W, lin_b):
    src = edge_index[0]
    dst = edge_index[1]
    x = jnp.take(emb, z, axis=0)
    diff = jnp.take(pos, dst, axis=0) - jnp.take(pos, src, axis=0)
    d2 = jnp.sum(diff * diff, axis=-1, keepdims=True)

    for b in range(NBLK):
        h = _node_lin(x, lin1_W[b].T)
        hg = jnp.take(h, src, axis=0)
        msg = _edge_msg(d2, hg, f1_W[b].T, f1_b[b][None, :], f2_W[b].T,
                        f2_b[b][None, :])
        agg = jnp.zeros((N, H), jnp.float32).at[dst].add(msg)
        x = _node_update(x, agg, lin2_W[b].T, lin2_b[b][None, :], lin_W[b].T,
                         lin_b[b][None, :])
    return x


# speedup vs baseline: 1.2601x; 1.2601x over previous
import functools

import jax
import jax.numpy as jnp
from jax import lax
from jax.experimental import pallas as pl
from jax.experimental.pallas import tpu as pltpu
from jax.experimental.pallas import tpu_sc as plsc

N = 10000
E = 320000
H = 128
NRBF = 50
NBLK = 3
CUTOFF = 5.0

EB = 5000

SC_CORES = 2
SC_SUBCORES = 16
SC_TILES = SC_CORES * SC_SUBCORES
CHUNK = 128
NCHUNK = E // CHUNK
ROWS_PER_SUB = N // SC_SUBCORES
ZCH = 400


def _sc_mesh():
    return plsc.VectorSubcoreMesh(core_axis_name="c", subcore_axis_name="s")


@jax.jit
def _sc_scatter_add(msg, dst, zeros):
    @functools.partial(
        pl.kernel,
        mesh=_sc_mesh(),
        out_type=jax.ShapeDtypeStruct((SC_CORES, N, H), jnp.float32),
        scratch_types=[
            pltpu.VMEM_SHARED((N, H), jnp.float32),
            pltpu.VMEM((1, CHUNK), jnp.int32),
            pltpu.VMEM((CHUNK, H), jnp.float32),
        ],
    )
    def k(msg_hbm, dst_hbm, zeros_hbm, out_hbm, shared, idx_v, msg_v):
        c = lax.axis_index("c")
        s = lax.axis_index("s")
        wid = s * SC_CORES + c

        nzc = N // ZCH

        @pl.loop(0, (nzc + SC_SUBCORES - 1) // SC_SUBCORES)
        def _(k):
            zc = s + k * SC_SUBCORES

            @pl.when(zc < nzc)
            def _():
                pltpu.sync_copy(zeros_hbm.at[pl.ds(zc * ZCH, ZCH)],
                                shared.at[pl.ds(zc * ZCH, ZCH)])

        plsc.subcore_barrier()

        nfull = NCHUNK // SC_TILES

        @pl.loop(0, nfull + 1)
        def _(i):
            cid = wid + i * SC_TILES

            @pl.when(cid < NCHUNK)
            def _():
                base = cid * CHUNK
                pltpu.sync_copy(dst_hbm.at[pl.ds(base, CHUNK)], idx_v.at[0])
                pltpu.sync_copy(msg_hbm.at[pl.ds(base, CHUNK)], msg_v)
                pltpu.sync_copy(msg_v, shared.at[idx_v.at[0]], add=True)

        plsc.subcore_barrier()

        @pl.loop(0, (nzc + SC_SUBCORES - 1) // SC_SUBCORES)
        def _(k):
            zc = s + k * SC_SUBCORES

            @pl.when(zc < nzc)
            def _():
                pltpu.sync_copy(shared.at[pl.ds(zc * ZCH, ZCH)],
                                out_hbm.at[c].at[pl.ds(zc * ZCH, ZCH)])

    return k(msg, dst, zeros)


_LOG2 = 0.6931471805599453


def _softplus(x):
    return jnp.maximum(x, 0.0) + jnp.log(1.0 + jnp.exp(-jnp.abs(x))) - _LOG2


def _edge_msg_body(d2_ref, hg_ref, f1_wt_ref, f1_b_ref, f2_wt_ref, f2_b_ref,
                   out_ref):
    d2 = d2_ref[...]
    d = jnp.sqrt(d2 + 1e-12)
    delta = CUTOFF / (NRBF - 1)
    offs = jax.lax.broadcasted_iota(jnp.int32, (EB, NRBF), 1).astype(
        jnp.float32) * delta
    coeff = -0.5 / (delta * delta)
    rbf = jnp.exp(coeff * (d - offs) ** 2)
    w = _softplus(jnp.dot(rbf, f1_wt_ref[...],
                          preferred_element_type=jnp.float32) + f1_b_ref[...])
    w = _softplus(jnp.dot(w, f2_wt_ref[...],
                          preferred_element_type=jnp.float32) + f2_b_ref[...])
    c = 0.5 * (jnp.cos(d * (jnp.pi / CUTOFF)) + 1.0)
    c = jnp.where(d < CUTOFF, c, 0.0)
    out_ref[...] = hg_ref[...] * w * c


@jax.jit
def _edge_msg(d2, hg, f1_wt, f1_b, f2_wt, f2_b):
    grid = (E // EB,)
    return pl.pallas_call(
        _edge_msg_body,
        grid=grid,
        in_specs=[
            pl.BlockSpec((EB, 1), lambda i: (i, 0)),
            pl.BlockSpec((EB, H), lambda i: (i, 0)),
            pl.BlockSpec((NRBF, H), lambda i: (0, 0)),
            pl.BlockSpec((1, H), lambda i: (0, 0)),
            pl.BlockSpec((H, H), lambda i: (0, 0)),
            pl.BlockSpec((1, H), lambda i: (0, 0)),
        ],
        out_specs=pl.BlockSpec((EB, H), lambda i: (i, 0)),
        out_shape=jax.ShapeDtypeStruct((E, H), jnp.float32),
    )(d2, hg, f1_wt, f1_b, f2_wt, f2_b)


def _node_lin_body(x_ref, wt_ref, out_ref):
    out_ref[...] = jnp.dot(x_ref[...], wt_ref[...],
                           preferred_element_type=jnp.float32)


@jax.jit
def _node_lin(x, wt):
    return pl.pallas_call(
        _node_lin_body,
        out_shape=jax.ShapeDtypeStruct((N, H), jnp.float32),
    )(x, wt)


def _node_update_body(x_ref, agg0_ref, agg1_ref, lin2_wt_ref, lin2_b_ref,
                      lin_wt_ref, lin_b_ref, out_ref):
    agg = agg0_ref[...] + agg1_ref[...]
    h = jnp.dot(agg, lin2_wt_ref[...],
                preferred_element_type=jnp.float32) + lin2_b_ref[...]
    h = jnp.tanh(h)
    h = jnp.dot(h, lin_wt_ref[...],
                preferred_element_type=jnp.float32) + lin_b_ref[...]
    out_ref[...] = x_ref[...] + h


@jax.jit
def _node_update(x, agg0, agg1, lin2_wt, lin2_b, lin_wt, lin_b):
    return pl.pallas_call(
        _node_update_body,
        out_shape=jax.ShapeDtypeStruct((N, H), jnp.float32),
    )(x, agg0, agg1, lin2_wt, lin2_b, lin_wt, lin_b)


def kernel(z, pos, edge_index, emb, lin1_W, f1_W, f1_b, f2_W, f2_b, lin2_W,
           lin2_b, lin_W, lin_b):
    src = edge_index[0]
    dst = edge_index[1].astype(jnp.int32)
    x = jnp.take(emb, z, axis=0)
    diff = jnp.take(pos, dst, axis=0) - jnp.take(pos, src, axis=0)
    d2 = jnp.sum(diff * diff, axis=-1, keepdims=True)
    zeros = jnp.zeros((N, H), jnp.float32)

    for b in range(NBLK):
        h = _node_lin(x, lin1_W[b].T)
        hg = jnp.take(h, src, axis=0)
        msg = _edge_msg(d2, hg, f1_W[b].T, f1_b[b][None, :], f2_W[b].T,
                        f2_b[b][None, :])
        aggs = _sc_scatter_add(msg, dst, zeros)
        x = _node_update(x, aggs[0], aggs[1], lin2_W[b].T, lin2_b[b][None, :],
                         lin_W[b].T, lin_b[b][None, :])
    return x


# speedup vs baseline: 1.7144x; 1.3605x over previous
import functools

import jax
import jax.numpy as jnp
from jax import lax
from jax.experimental import pallas as pl
from jax.experimental.pallas import tpu as pltpu
from jax.experimental.pallas import tpu_sc as plsc

N = 10000
E = 320000
H = 128
NRBF = 50
NBLK = 3
CUTOFF = 5.0

EB = 5000

SC_CORES = 2
SC_SUBCORES = 16
SC_TILES = SC_CORES * SC_SUBCORES
CHUNK = 128
NCHUNK = E // CHUNK
ROWS_PER_SUB = N // SC_SUBCORES
ZCH = 400


def _sc_mesh():
    return plsc.VectorSubcoreMesh(core_axis_name="c", subcore_axis_name="s")


@jax.jit
def _sc_scatter_add(msg, dst, zeros):
    @functools.partial(
        pl.kernel,
        mesh=_sc_mesh(),
        out_type=jax.ShapeDtypeStruct((SC_CORES, N, H), jnp.float32),
        scratch_types=[
            pltpu.VMEM_SHARED((N, H), jnp.float32),
            pltpu.VMEM((1, CHUNK), jnp.int32),
            pltpu.VMEM((CHUNK, H), jnp.float32),
        ],
    )
    def k(msg_hbm, dst_hbm, zeros_hbm, out_hbm, shared, idx_v, msg_v):
        c = lax.axis_index("c")
        s = lax.axis_index("s")
        wid = s * SC_CORES + c

        nzc = N // ZCH

        @pl.loop(0, (nzc + SC_SUBCORES - 1) // SC_SUBCORES)
        def _(k):
            zc = s + k * SC_SUBCORES

            @pl.when(zc < nzc)
            def _():
                pltpu.sync_copy(zeros_hbm.at[pl.ds(zc * ZCH, ZCH)],
                                shared.at[pl.ds(zc * ZCH, ZCH)])

        plsc.subcore_barrier()

        nfull = NCHUNK // SC_TILES

        @pl.loop(0, nfull + 1)
        def _(i):
            cid = wid + i * SC_TILES

            @pl.when(cid < NCHUNK)
            def _():
                base = cid * CHUNK
                pltpu.sync_copy(dst_hbm.at[pl.ds(base, CHUNK)], idx_v.at[0])
                pltpu.sync_copy(msg_hbm.at[pl.ds(base, CHUNK)], msg_v)
                pltpu.sync_copy(msg_v, shared.at[idx_v.at[0]], add=True)

        plsc.subcore_barrier()

        @pl.loop(0, (nzc + SC_SUBCORES - 1) // SC_SUBCORES)
        def _(k):
            zc = s + k * SC_SUBCORES

            @pl.when(zc < nzc)
            def _():
                pltpu.sync_copy(shared.at[pl.ds(zc * ZCH, ZCH)],
                                out_hbm.at[c].at[pl.ds(zc * ZCH, ZCH)])

    return k(msg, dst, zeros)


_LOG2 = 0.6931471805599453


def _softplus(x):
    return jnp.maximum(x, 0.0) + jnp.log(1.0 + jnp.exp(-jnp.abs(x))) - _LOG2


@jax.jit
def _sc_gather(table, idx):
    @functools.partial(
        pl.kernel,
        mesh=_sc_mesh(),
        out_type=jax.ShapeDtypeStruct((E, H), jnp.float32),
        scratch_types=[
            pltpu.VMEM((1, CHUNK), jnp.int32),
            pltpu.VMEM((CHUNK, H), jnp.float32),
        ],
    )
    def k(table_hbm, idx_hbm, out_hbm, idx_v, rows_v):
        c = lax.axis_index("c")
        s = lax.axis_index("s")
        wid = s * SC_CORES + c

        nfull = NCHUNK // SC_TILES

        @pl.loop(0, nfull + 1)
        def _(i):
            cid = wid + i * SC_TILES

            @pl.when(cid < NCHUNK)
            def _():
                base = cid * CHUNK
                pltpu.sync_copy(idx_hbm.at[pl.ds(base, CHUNK)], idx_v.at[0])
                pltpu.sync_copy(table_hbm.at[idx_v.at[0]], rows_v)
                pltpu.sync_copy(rows_v, out_hbm.at[pl.ds(base, CHUNK)])

    return k(table, idx)


def _edge_msg_body(d2_ref, hg_ref, f1_wt_ref, f1_b_ref, f2_wt_ref, f2_b_ref,
                   out_ref):
    d2 = d2_ref[...]
    d = jnp.sqrt(d2 + 1e-12)
    delta = CUTOFF / (NRBF - 1)
    offs = jax.lax.broadcasted_iota(jnp.int32, (EB, NRBF), 1).astype(
        jnp.float32) * delta
    coeff = -0.5 / (delta * delta)
    rbf = jnp.exp(coeff * (d - offs) ** 2)
    w = _softplus(jnp.dot(rbf, f1_wt_ref[...],
                          preferred_element_type=jnp.float32) + f1_b_ref[...])
    w = _softplus(jnp.dot(w, f2_wt_ref[...],
                          preferred_element_type=jnp.float32) + f2_b_ref[...])
    c = 0.5 * (jnp.cos(d * (jnp.pi / CUTOFF)) + 1.0)
    c = jnp.where(d < CUTOFF, c, 0.0)
    out_ref[...] = hg_ref[...] * w * c


@jax.jit
def _edge_msg(d2, hg, f1_wt, f1_b, f2_wt, f2_b):
    grid = (E // EB,)
    return pl.pallas_call(
        _edge_msg_body,
        grid=grid,
        in_specs=[
            pl.BlockSpec((EB, 1), lambda i: (i, 0)),
            pl.BlockSpec((EB, H), lambda i: (i, 0)),
            pl.BlockSpec((NRBF, H), lambda i: (0, 0)),
            pl.BlockSpec((1, H), lambda i: (0, 0)),
            pl.BlockSpec((H, H), lambda i: (0, 0)),
            pl.BlockSpec((1, H), lambda i: (0, 0)),
        ],
        out_specs=pl.BlockSpec((EB, H), lambda i: (i, 0)),
        out_shape=jax.ShapeDtypeStruct((E, H), jnp.float32),
    )(d2, hg, f1_wt, f1_b, f2_wt, f2_b)


def _node_lin_body(x_ref, wt_ref, out_ref):
    out_ref[...] = jnp.dot(x_ref[...], wt_ref[...],
                           preferred_element_type=jnp.float32)


@jax.jit
def _node_lin(x, wt):
    return pl.pallas_call(
        _node_lin_body,
        out_shape=jax.ShapeDtypeStruct((N, H), jnp.float32),
    )(x, wt)


def _node_update_body(x_ref, agg0_ref, agg1_ref, lin2_wt_ref, lin2_b_ref,
                      lin_wt_ref, lin_b_ref, out_ref):
    agg = agg0_ref[...] + agg1_ref[...]
    h = jnp.dot(agg, lin2_wt_ref[...],
                preferred_element_type=jnp.float32) + lin2_b_ref[...]
    h = jnp.tanh(h)
    h = jnp.dot(h, lin_wt_ref[...],
                preferred_element_type=jnp.float32) + lin_b_ref[...]
    out_ref[...] = x_ref[...] + h


@jax.jit
def _node_update(x, agg0, agg1, lin2_wt, lin2_b, lin_wt, lin_b):
    return pl.pallas_call(
        _node_update_body,
        out_shape=jax.ShapeDtypeStruct((N, H), jnp.float32),
    )(x, agg0, agg1, lin2_wt, lin2_b, lin_wt, lin_b)


def kernel(z, pos, edge_index, emb, lin1_W, f1_W, f1_b, f2_W, f2_b, lin2_W,
           lin2_b, lin_W, lin_b):
    src = edge_index[0].astype(jnp.int32)
    dst = edge_index[1].astype(jnp.int32)
    x = jnp.take(emb, z, axis=0)
    diff = jnp.take(pos, dst, axis=0) - jnp.take(pos, src, axis=0)
    d2 = jnp.sum(diff * diff, axis=-1, keepdims=True)
    zeros = jnp.zeros((N, H), jnp.float32)

    for b in range(NBLK):
        h = _node_lin(x, lin1_W[b].T)
        hg = _sc_gather(h, src)
        msg = _edge_msg(d2, hg, f1_W[b].T, f1_b[b][None, :], f2_W[b].T,
                        f2_b[b][None, :])
        aggs = _sc_scatter_add(msg, dst, zeros)
        x = _node_update(x, aggs[0], aggs[1], lin2_W[b].T, lin2_b[b][None, :],
                         lin_W[b].T, lin_b[b][None, :])
    return x


# speedup vs baseline: 2.4259x; 1.4150x over previous
import dataclasses
import functools

import jax
import jax.numpy as jnp
from jax import lax
from jax.experimental import pallas as pl
from jax.experimental.pallas import tpu as pltpu
from jax.experimental.pallas import tpu_sc as plsc

N = 10000
E = 320000
H = 128
NRBF = 50
NBLK = 3
CUTOFF = 5.0

EB = 5000

SC_CORES = 2
SC_SUBCORES = 16
SC_TILES = SC_CORES * SC_SUBCORES
CHUNK = 128
NCHUNK = E // CHUNK
ROWS_PER_SUB = N // SC_SUBCORES
ZCH = 400


def _sc_mesh():
    return plsc.VectorSubcoreMesh(core_axis_name="c", subcore_axis_name="s")


def _sc_compiler_params():
    cp = pltpu.CompilerParams()
    if "needs_layout_passes" in pltpu.CompilerParams.__dataclass_fields__:
        cp = dataclasses.replace(cp, needs_layout_passes=False)
    return cp


@jax.jit
def _sc_scatter_add(msg, dst, zeros):
    @functools.partial(
        pl.kernel,
        mesh=_sc_mesh(),
        out_type=jax.ShapeDtypeStruct((SC_CORES, N, H), jnp.float32),
        scratch_types=[
            pltpu.VMEM_SHARED((N, H), jnp.float32),
            pltpu.VMEM((1, CHUNK), jnp.int32),
            pltpu.VMEM((CHUNK, H), jnp.float32),
        ],
    )
    def k(msg_hbm, dst_hbm, zeros_hbm, out_hbm, shared, idx_v, msg_v):
        c = lax.axis_index("c")
        s = lax.axis_index("s")
        wid = s * SC_CORES + c

        nzc = N // ZCH

        @pl.loop(0, (nzc + SC_SUBCORES - 1) // SC_SUBCORES)
        def _(k):
            zc = s + k * SC_SUBCORES

            @pl.when(zc < nzc)
            def _():
                pltpu.sync_copy(zeros_hbm.at[pl.ds(zc * ZCH, ZCH)],
                                shared.at[pl.ds(zc * ZCH, ZCH)])

        plsc.subcore_barrier()

        nfull = NCHUNK // SC_TILES

        @pl.loop(0, nfull + 1)
        def _(i):
            cid = wid + i * SC_TILES

            @pl.when(cid < NCHUNK)
            def _():
                base = cid * CHUNK
                pltpu.sync_copy(dst_hbm.at[pl.ds(base, CHUNK)], idx_v.at[0])
                pltpu.sync_copy(msg_hbm.at[pl.ds(base, CHUNK)], msg_v)
                pltpu.sync_copy(msg_v, shared.at[idx_v.at[0]], add=True)

        plsc.subcore_barrier()

        @pl.loop(0, (nzc + SC_SUBCORES - 1) // SC_SUBCORES)
        def _(k):
            zc = s + k * SC_SUBCORES

            @pl.when(zc < nzc)
            def _():
                pltpu.sync_copy(shared.at[pl.ds(zc * ZCH, ZCH)],
                                out_hbm.at[c].at[pl.ds(zc * ZCH, ZCH)])

    return k(msg, dst, zeros)


_LOG2 = 0.6931471805599453


def _softplus(x):
    return jnp.maximum(x, 0.0) + jnp.log(1.0 + jnp.exp(-jnp.abs(x))) - _LOG2


EPT = E // SC_TILES


@jax.jit
def _sc_edge_d2(px, py, pz, src, dst):
    @functools.partial(
        pl.kernel,
        mesh=_sc_mesh(),
        out_type=jax.ShapeDtypeStruct((E,), jnp.float32),
        compiler_params=_sc_compiler_params(),
        scratch_types=[
            pltpu.VMEM((N,), jnp.float32),
            pltpu.VMEM((N,), jnp.float32),
            pltpu.VMEM((N,), jnp.float32),
            pltpu.VMEM((EPT,), jnp.int32),
            pltpu.VMEM((EPT,), jnp.int32),
            pltpu.VMEM((EPT,), jnp.float32),
        ],
    )
    def k(px_hbm, py_hbm, pz_hbm, src_hbm, dst_hbm, out_hbm,
          px_v, py_v, pz_v, src_v, dst_v, d2_v):
        c = lax.axis_index("c")
        s = lax.axis_index("s")
        wid = s * SC_CORES + c
        base = wid * EPT
        pltpu.sync_copy(px_hbm, px_v)
        pltpu.sync_copy(py_hbm, py_v)
        pltpu.sync_copy(pz_hbm, pz_v)
        pltpu.sync_copy(src_hbm.at[pl.ds(base, EPT)], src_v)
        pltpu.sync_copy(dst_hbm.at[pl.ds(base, EPT)], dst_v)

        @pl.loop(0, EPT // 16)
        def _(i):
            o = i * 16
            si = src_v[pl.ds(o, 16)]
            di = dst_v[pl.ds(o, 16)]
            dx = plsc.load_gather(px_v, [di]) - plsc.load_gather(px_v, [si])
            dy = plsc.load_gather(py_v, [di]) - plsc.load_gather(py_v, [si])
            dz = plsc.load_gather(pz_v, [di]) - plsc.load_gather(pz_v, [si])
            d2_v[pl.ds(o, 16)] = dx * dx + dy * dy + dz * dz

        pltpu.sync_copy(d2_v, out_hbm.at[pl.ds(base, EPT)])

    return k(px, py, pz, src, dst)


@jax.jit
def _sc_gather(table, idx):
    @functools.partial(
        pl.kernel,
        mesh=_sc_mesh(),
        out_type=jax.ShapeDtypeStruct((E, H), jnp.float32),
        scratch_types=[
            pltpu.VMEM((1, CHUNK), jnp.int32),
            pltpu.VMEM((CHUNK, H), jnp.float32),
        ],
    )
    def k(table_hbm, idx_hbm, out_hbm, idx_v, rows_v):
        c = lax.axis_index("c")
        s = lax.axis_index("s")
        wid = s * SC_CORES + c

        nfull = NCHUNK // SC_TILES

        @pl.loop(0, nfull + 1)
        def _(i):
            cid = wid + i * SC_TILES

            @pl.when(cid < NCHUNK)
            def _():
                base = cid * CHUNK
                pltpu.sync_copy(idx_hbm.at[pl.ds(base, CHUNK)], idx_v.at[0])
                pltpu.sync_copy(table_hbm.at[idx_v.at[0]], rows_v)
                pltpu.sync_copy(rows_v, out_hbm.at[pl.ds(base, CHUNK)])

    return k(table, idx)


def _edge_msg_body(d2_ref, hg_ref, f1_wt_ref, f1_b_ref, f2_wt_ref, f2_b_ref,
                   out_ref):
    d2 = d2_ref[...]
    d = jnp.sqrt(d2 + 1e-12)
    delta = CUTOFF / (NRBF - 1)
    offs = jax.lax.broadcasted_iota(jnp.int32, (EB, NRBF), 1).astype(
        jnp.float32) * delta
    coeff = -0.5 / (delta * delta)
    rbf = jnp.exp(coeff * (d - offs) ** 2)
    w = _softplus(jnp.dot(rbf, f1_wt_ref[...],
                          preferred_element_type=jnp.float32) + f1_b_ref[...])
    w = _softplus(jnp.dot(w, f2_wt_ref[...],
                          preferred_element_type=jnp.float32) + f2_b_ref[...])
    c = 0.5 * (jnp.cos(d * (jnp.pi / CUTOFF)) + 1.0)
    c = jnp.where(d < CUTOFF, c, 0.0)
    out_ref[...] = hg_ref[...] * w * c


@jax.jit
def _edge_msg(d2, hg, f1_wt, f1_b, f2_wt, f2_b):
    grid = (E // EB,)
    return pl.pallas_call(
        _edge_msg_body,
        grid=grid,
        in_specs=[
            pl.BlockSpec((EB, 1), lambda i: (i, 0)),
            pl.BlockSpec((EB, H), lambda i: (i, 0)),
            pl.BlockSpec((NRBF, H), lambda i: (0, 0)),
            pl.BlockSpec((1, H), lambda i: (0, 0)),
            pl.BlockSpec((H, H), lambda i: (0, 0)),
            pl.BlockSpec((1, H), lambda i: (0, 0)),
        ],
        out_specs=pl.BlockSpec((EB, H), lambda i: (i, 0)),
        out_shape=jax.ShapeDtypeStruct((E, H), jnp.float32),
    )(d2, hg, f1_wt, f1_b, f2_wt, f2_b)


def _node_lin_body(x_ref, wt_ref, out_ref):
    out_ref[...] = jnp.dot(x_ref[...], wt_ref[...],
                           preferred_element_type=jnp.float32)


@jax.jit
def _node_lin(x, wt):
    return pl.pallas_call(
        _node_lin_body,
        out_shape=jax.ShapeDtypeStruct((N, H), jnp.float32),
    )(x, wt)


def _node_update_body(x_ref, agg0_ref, agg1_ref, lin2_wt_ref, lin2_b_ref,
                      lin_wt_ref, lin_b_ref, out_ref):
    agg = agg0_ref[...] + agg1_ref[...]
    h = jnp.dot(agg, lin2_wt_ref[...],
                preferred_element_type=jnp.float32) + lin2_b_ref[...]
    h = jnp.tanh(h)
    h = jnp.dot(h, lin_wt_ref[...],
                preferred_element_type=jnp.float32) + lin_b_ref[...]
    out_ref[...] = x_ref[...] + h


@jax.jit
def _node_update(x, agg0, agg1, lin2_wt, lin2_b, lin_wt, lin_b):
    return pl.pallas_call(
        _node_update_body,
        out_shape=jax.ShapeDtypeStruct((N, H), jnp.float32),
    )(x, agg0, agg1, lin2_wt, lin2_b, lin_wt, lin_b)


def kernel(z, pos, edge_index, emb, lin1_W, f1_W, f1_b, f2_W, f2_b, lin2_W,
           lin2_b, lin_W, lin_b):
    src = edge_index[0].astype(jnp.int32)
    dst = edge_index[1].astype(jnp.int32)
    x = jnp.take(emb, z, axis=0)
    pt = pos.T
    d2 = _sc_edge_d2(pt[0], pt[1], pt[2], src, dst)[:, None]
    zeros = jnp.zeros((N, H), jnp.float32)

    for b in range(NBLK):
        h = _node_lin(x, lin1_W[b].T)
        hg = _sc_gather(h, src)
        msg = _edge_msg(d2, hg, f1_W[b].T, f1_b[b][None, :], f2_W[b].T,
                        f2_b[b][None, :])
        aggs = _sc_scatter_add(msg, dst, zeros)
        x = _node_update(x, aggs[0], aggs[1], lin2_W[b].T, lin2_b[b][None, :],
                         lin_W[b].T, lin_b[b][None, :])
    return x


# speedup vs baseline: 5.5838x; 2.3018x over previous
import dataclasses
import functools

import jax
import jax.numpy as jnp
from jax import lax
from jax.experimental import pallas as pl
from jax.experimental.pallas import tpu as pltpu
from jax.experimental.pallas import tpu_sc as plsc

N = 10000
E = 320000
H = 128
NRBF = 50
NBLK = 3
CUTOFF = 5.0

EB = 5000

SC_CORES = 2
SC_SUBCORES = 16
SC_TILES = SC_CORES * SC_SUBCORES
CHUNK = 128
NCHUNK = E // CHUNK
ROWS_PER_SUB = N // SC_SUBCORES
ZCH = 400


def _sc_mesh():
    return plsc.VectorSubcoreMesh(core_axis_name="c", subcore_axis_name="s")


def _sc_compiler_params():
    cp = pltpu.CompilerParams()
    if "needs_layout_passes" in pltpu.CompilerParams.__dataclass_fields__:
        cp = dataclasses.replace(cp, needs_layout_passes=False)
    return cp


@jax.jit
def _sc_scatter_add(msg, dst, zeros):
    @functools.partial(
        pl.kernel,
        mesh=_sc_mesh(),
        out_type=jax.ShapeDtypeStruct((SC_CORES, N, H), jnp.float32),
        scratch_types=[
            pltpu.VMEM_SHARED((N, H), jnp.float32),
            pltpu.VMEM((1, CHUNK), jnp.int32),
            pltpu.VMEM((CHUNK, H), jnp.float32),
        ],
    )
    def k(msg_hbm, dst_hbm, zeros_hbm, out_hbm, shared, idx_v, msg_v):
        c = lax.axis_index("c")
        s = lax.axis_index("s")
        wid = s * SC_CORES + c

        nzc = N // ZCH

        @pl.loop(0, (nzc + SC_SUBCORES - 1) // SC_SUBCORES)
        def _(k):
            zc = s + k * SC_SUBCORES

            @pl.when(zc < nzc)
            def _():
                pltpu.sync_copy(zeros_hbm.at[pl.ds(zc * ZCH, ZCH)],
                                shared.at[pl.ds(zc * ZCH, ZCH)])

        plsc.subcore_barrier()

        nfull = NCHUNK // SC_TILES

        @pl.loop(0, nfull + 1)
        def _(i):
            cid = wid + i * SC_TILES

            @pl.when(cid < NCHUNK)
            def _():
                base = cid * CHUNK
                pltpu.sync_copy(dst_hbm.at[pl.ds(base, CHUNK)], idx_v.at[0])
                pltpu.sync_copy(msg_hbm.at[pl.ds(base, CHUNK)], msg_v)
                pltpu.sync_copy(msg_v, shared.at[idx_v.at[0]], add=True)

        plsc.subcore_barrier()

        @pl.loop(0, (nzc + SC_SUBCORES - 1) // SC_SUBCORES)
        def _(k):
            zc = s + k * SC_SUBCORES

            @pl.when(zc < nzc)
            def _():
                pltpu.sync_copy(shared.at[pl.ds(zc * ZCH, ZCH)],
                                out_hbm.at[c].at[pl.ds(zc * ZCH, ZCH)])

    return k(msg, dst, zeros)


_LOG2 = 0.6931471805599453


def _softplus(x):
    return jnp.maximum(x, 0.0) + jnp.log(1.0 + jnp.exp(-jnp.abs(x))) - _LOG2


EPT = E // SC_TILES

TKNOT = 8192
DINV = TKNOT / 8.0
TB = 2048


def _table_body(f1_wt_ref, f1_b_ref, f2_wt_ref, f2_b_ref, out_ref):
    i = pl.program_id(0)
    r = jax.lax.broadcasted_iota(jnp.int32, (TB, 1), 0) + i * TB
    d = r.astype(jnp.float32) * (1.0 / DINV)
    delta = CUTOFF / (NRBF - 1)
    offs = jax.lax.broadcasted_iota(jnp.int32, (TB, NRBF), 1).astype(
        jnp.float32) * delta
    coeff = -0.5 / (delta * delta)
    rbf = jnp.exp(coeff * (d - offs) ** 2)
    w = _softplus(jnp.dot(rbf, f1_wt_ref[...],
                          preferred_element_type=jnp.float32) + f1_b_ref[...])
    w = _softplus(jnp.dot(w, f2_wt_ref[...],
                          preferred_element_type=jnp.float32) + f2_b_ref[...])
    c = 0.5 * (jnp.cos(d * (jnp.pi / CUTOFF)) + 1.0)
    c = jnp.where(d < CUTOFF, c, 0.0)
    out_ref[...] = w * c


@jax.jit
def _filter_table(f1_wt, f1_b, f2_wt, f2_b):
    return pl.pallas_call(
        _table_body,
        grid=(TKNOT // TB,),
        in_specs=[
            pl.BlockSpec((NRBF, H), lambda i: (0, 0)),
            pl.BlockSpec((1, H), lambda i: (0, 0)),
            pl.BlockSpec((H, H), lambda i: (0, 0)),
            pl.BlockSpec((1, H), lambda i: (0, 0)),
        ],
        out_specs=pl.BlockSpec((TB, H), lambda i: (i, 0)),
        out_shape=jax.ShapeDtypeStruct((TKNOT, H), jnp.float32),
    )(f1_wt, f1_b, f2_wt, f2_b)


def _edge_q_body(d2_ref, q_ref):
    d = jnp.sqrt(d2_ref[...] + 1e-12)
    q = jnp.floor(d * DINV + 0.5).astype(jnp.int32)
    q_ref[...] = jnp.clip(q, 0, TKNOT - 1)


@jax.jit
def _edge_q(d2_flat):
    d2m = d2_flat.reshape(E // H, H)
    q = pl.pallas_call(
        _edge_q_body,
        out_shape=jax.ShapeDtypeStruct((E // H, H), jnp.int32),
    )(d2m)
    return q.reshape(E)


@jax.jit
def _sc_gather_mul_scatter(g_tab, h, q, src, dst, zeros):
    @functools.partial(
        pl.kernel,
        mesh=_sc_mesh(),
        out_type=jax.ShapeDtypeStruct((SC_CORES, N, H), jnp.float32),
        compiler_params=_sc_compiler_params(),
        scratch_types=[
            pltpu.VMEM_SHARED((N, H), jnp.float32),
            pltpu.VMEM((1, CHUNK), jnp.int32),
            pltpu.VMEM((1, CHUNK), jnp.int32),
            pltpu.VMEM((1, CHUNK), jnp.int32),
            pltpu.VMEM((CHUNK, H), jnp.float32),
            pltpu.VMEM((CHUNK, H), jnp.float32),
        ],
    )
    def k(g_hbm, h_hbm, q_hbm, src_hbm, dst_hbm, zeros_hbm, out_hbm,
          shared, qi_v, si_v, di_v, g_v, h_v):
        c = lax.axis_index("c")
        s = lax.axis_index("s")
        wid = s * SC_CORES + c

        nzc = N // ZCH

        @pl.loop(0, (nzc + SC_SUBCORES - 1) // SC_SUBCORES)
        def _(kk):
            zc = s + kk * SC_SUBCORES

            @pl.when(zc < nzc)
            def _():
                pltpu.sync_copy(zeros_hbm.at[pl.ds(zc * ZCH, ZCH)],
                                shared.at[pl.ds(zc * ZCH, ZCH)])

        plsc.subcore_barrier()

        nfull = NCHUNK // SC_TILES

        @pl.loop(0, nfull + 1)
        def _(i):
            cid = wid + i * SC_TILES

            @pl.when(cid < NCHUNK)
            def _():
                base = cid * CHUNK
                pltpu.sync_copy(q_hbm.at[pl.ds(base, CHUNK)], qi_v.at[0])
                pltpu.sync_copy(src_hbm.at[pl.ds(base, CHUNK)], si_v.at[0])
                pltpu.sync_copy(dst_hbm.at[pl.ds(base, CHUNK)], di_v.at[0])
                pltpu.sync_copy(g_hbm.at[qi_v.at[0]], g_v)
                pltpu.sync_copy(h_hbm.at[si_v.at[0]], h_v)

                @pl.loop(0, CHUNK)
                def _(r):
                    for kk in range(H // 16):
                        sl = pl.ds(kk * 16, 16)
                        g_v[r, sl] = g_v[r, sl] * h_v[r, sl]

                pltpu.sync_copy(g_v, shared.at[di_v.at[0]], add=True)

        plsc.subcore_barrier()

        @pl.loop(0, (nzc + SC_SUBCORES - 1) // SC_SUBCORES)
        def _(kk):
            zc = s + kk * SC_SUBCORES

            @pl.when(zc < nzc)
            def _():
                pltpu.sync_copy(shared.at[pl.ds(zc * ZCH, ZCH)],
                                out_hbm.at[c].at[pl.ds(zc * ZCH, ZCH)])

    return k(g_tab, h, q, src, dst, zeros)


@jax.jit
def _sc_edge_d2(px, py, pz, src, dst):
    @functools.partial(
        pl.kernel,
        mesh=_sc_mesh(),
        out_type=jax.ShapeDtypeStruct((E,), jnp.float32),
        compiler_params=_sc_compiler_params(),
        scratch_types=[
            pltpu.VMEM((N,), jnp.float32),
            pltpu.VMEM((N,), jnp.float32),
            pltpu.VMEM((N,), jnp.float32),
            pltpu.VMEM((EPT,), jnp.int32),
            pltpu.VMEM((EPT,), jnp.int32),
            pltpu.VMEM((EPT,), jnp.float32),
        ],
    )
    def k(px_hbm, py_hbm, pz_hbm, src_hbm, dst_hbm, out_hbm,
          px_v, py_v, pz_v, src_v, dst_v, d2_v):
        c = lax.axis_index("c")
        s = lax.axis_index("s")
        wid = s * SC_CORES + c
        base = wid * EPT
        pltpu.sync_copy(px_hbm, px_v)
        pltpu.sync_copy(py_hbm, py_v)
        pltpu.sync_copy(pz_hbm, pz_v)
        pltpu.sync_copy(src_hbm.at[pl.ds(base, EPT)], src_v)
        pltpu.sync_copy(dst_hbm.at[pl.ds(base, EPT)], dst_v)

        @pl.loop(0, EPT // 16)
        def _(i):
            o = i * 16
            si = src_v[pl.ds(o, 16)]
            di = dst_v[pl.ds(o, 16)]
            dx = plsc.load_gather(px_v, [di]) - plsc.load_gather(px_v, [si])
            dy = plsc.load_gather(py_v, [di]) - plsc.load_gather(py_v, [si])
            dz = plsc.load_gather(pz_v, [di]) - plsc.load_gather(pz_v, [si])
            d2_v[pl.ds(o, 16)] = dx * dx + dy * dy + dz * dz

        pltpu.sync_copy(d2_v, out_hbm.at[pl.ds(base, EPT)])

    return k(px, py, pz, src, dst)


@jax.jit
def _sc_gather(table, idx):
    @functools.partial(
        pl.kernel,
        mesh=_sc_mesh(),
        out_type=jax.ShapeDtypeStruct((E, H), jnp.float32),
        scratch_types=[
            pltpu.VMEM((1, CHUNK), jnp.int32),
            pltpu.VMEM((CHUNK, H), jnp.float32),
        ],
    )
    def k(table_hbm, idx_hbm, out_hbm, idx_v, rows_v):
        c = lax.axis_index("c")
        s = lax.axis_index("s")
        wid = s * SC_CORES + c

        nfull = NCHUNK // SC_TILES

        @pl.loop(0, nfull + 1)
        def _(i):
            cid = wid + i * SC_TILES

            @pl.when(cid < NCHUNK)
            def _():
                base = cid * CHUNK
                pltpu.sync_copy(idx_hbm.at[pl.ds(base, CHUNK)], idx_v.at[0])
                pltpu.sync_copy(table_hbm.at[idx_v.at[0]], rows_v)
                pltpu.sync_copy(rows_v, out_hbm.at[pl.ds(base, CHUNK)])

    return k(table, idx)


def _edge_msg_body(d2_ref, hg_ref, f1_wt_ref, f1_b_ref, f2_wt_ref, f2_b_ref,
                   out_ref):
    d2 = d2_ref[...]
    d = jnp.sqrt(d2 + 1e-12)
    delta = CUTOFF / (NRBF - 1)
    offs = jax.lax.broadcasted_iota(jnp.int32, (EB, NRBF), 1).astype(
        jnp.float32) * delta
    coeff = -0.5 / (delta * delta)
    rbf = jnp.exp(coeff * (d - offs) ** 2)
    w = _softplus(jnp.dot(rbf, f1_wt_ref[...],
                          preferred_element_type=jnp.float32) + f1_b_ref[...])
    w = _softplus(jnp.dot(w, f2_wt_ref[...],
                          preferred_element_type=jnp.float32) + f2_b_ref[...])
    c = 0.5 * (jnp.cos(d * (jnp.pi / CUTOFF)) + 1.0)
    c = jnp.where(d < CUTOFF, c, 0.0)
    out_ref[...] = hg_ref[...] * w * c


@jax.jit
def _edge_msg(d2, hg, f1_wt, f1_b, f2_wt, f2_b):
    grid = (E // EB,)
    return pl.pallas_call(
        _edge_msg_body,
        grid=grid,
        in_specs=[
            pl.BlockSpec((EB, 1), lambda i: (i, 0)),
            pl.BlockSpec((EB, H), lambda i: (i, 0)),
            pl.BlockSpec((NRBF, H), lambda i: (0, 0)),
            pl.BlockSpec((1, H), lambda i: (0, 0)),
            pl.BlockSpec((H, H), lambda i: (0, 0)),
            pl.BlockSpec((1, H), lambda i: (0, 0)),
        ],
        out_specs=pl.BlockSpec((EB, H), lambda i: (i, 0)),
        out_shape=jax.ShapeDtypeStruct((E, H), jnp.float32),
    )(d2, hg, f1_wt, f1_b, f2_wt, f2_b)


def _node_lin_body(x_ref, wt_ref, out_ref):
    out_ref[...] = jnp.dot(x_ref[...], wt_ref[...],
                           preferred_element_type=jnp.float32)


@jax.jit
def _node_lin(x, wt):
    return pl.pallas_call(
        _node_lin_body,
        out_shape=jax.ShapeDtypeStruct((N, H), jnp.float32),
    )(x, wt)


def _node_update_body(x_ref, agg0_ref, agg1_ref, lin2_wt_ref, lin2_b_ref,
                      lin_wt_ref, lin_b_ref, out_ref):
    agg = agg0_ref[...] + agg1_ref[...]
    h = jnp.dot(agg, lin2_wt_ref[...],
                preferred_element_type=jnp.float32) + lin2_b_ref[...]
    h = jnp.tanh(h)
    h = jnp.dot(h, lin_wt_ref[...],
                preferred_element_type=jnp.float32) + lin_b_ref[...]
    out_ref[...] = x_ref[...] + h


@jax.jit
def _node_update(x, agg0, agg1, lin2_wt, lin2_b, lin_wt, lin_b):
    return pl.pallas_call(
        _node_update_body,
        out_shape=jax.ShapeDtypeStruct((N, H), jnp.float32),
    )(x, agg0, agg1, lin2_wt, lin2_b, lin_wt, lin_b)


def kernel(z, pos, edge_index, emb, lin1_W, f1_W, f1_b, f2_W, f2_b, lin2_W,
           lin2_b, lin_W, lin_b):
    src = edge_index[0].astype(jnp.int32)
    dst = edge_index[1].astype(jnp.int32)
    x = jnp.take(emb, z, axis=0)
    pt = pos.T
    d2 = _sc_edge_d2(pt[0], pt[1], pt[2], src, dst)
    q = _edge_q(d2)
    zeros = jnp.zeros((N, H), jnp.float32)

    for b in range(NBLK):
        g_tab = _filter_table(f1_W[b].T, f1_b[b][None, :], f2_W[b].T,
                              f2_b[b][None, :])
        h = _node_lin(x, lin1_W[b].T)
        aggs = _sc_gather_mul_scatter(g_tab, h, q, src, dst, zeros)
        x = _node_update(x, aggs[0], aggs[1], lin2_W[b].T, lin2_b[b][None, :],
                         lin_W[b].T, lin_b[b][None, :])
    return x
